# baseline (speedup 1.0000x reference)
import functools

import jax
import jax.numpy as jnp
from jax import lax
from jax.experimental import pallas as pl
from jax.experimental.pallas import tpu as pltpu
from jax.experimental.pallas import tpu_sc as plsc

_NC, _NS = 2, 16
_CH = 32
_LANES = 16


def _build_sc_kernel(B, LV, C, L, Lp):
    TROWS = LV + 1
    n_chunks = L // _CH
    per_half = n_chunks // 2
    tail = n_chunks * _CH
    ccq = C // _LANES // 8
    mesh = plsc.VectorSubcoreMesh(core_axis_name="c", subcore_axis_name="s")

    @functools.partial(
        pl.kernel,
        mesh=mesh,
        out_type=jax.ShapeDtypeStruct((B, L, C), jnp.float32),
        scratch_types=[
            pltpu.VMEM((Lp,), jnp.int32),
            pltpu.VMEM((Lp,), jnp.int32),
            pltpu.VMEM((_CH, C), jnp.float32),
            pltpu.VMEM((_CH, C), jnp.float32),
            pltpu.VMEM((_CH, C), jnp.float32),
            pltpu.VMEM((_CH, C), jnp.float32),
            pltpu.VMEM((8, C), jnp.float32),
            pltpu.VMEM((8, C), jnp.float32),
            pltpu.SemaphoreType.DMA,
            pltpu.SemaphoreType.DMA,
            pltpu.SemaphoreType.DMA,
            pltpu.SemaphoreType.DMA,
            pltpu.SemaphoreType.DMA,
            pltpu.SemaphoreType.DMA,
            pltpu.SemaphoreType.DMA,
            pltpu.SemaphoreType.DMA,
        ],
    )
    def sc_kernel(x_hbm, pol_hbm, pe_hbm, out_hbm, pol_v, idx_v,
                  pe0, g0, pe1, g1, pe_t, g_t,
                  spe0, sg0, so0, spe1, sg1, so1, spt, sgt):
        wid = lax.axis_index("s") * _NC + lax.axis_index("c")
        b = wid // 2
        h = wid % 2

        pltpu.sync_copy(pol_hbm.at[b], pol_v)

        bbase = b * TROWS
        lanes = lax.iota(jnp.int32, _LANES)

        def _take(v, i):
            return v.at[i].get(mode="promise_in_bounds")

        def scan_body(i, carry):
            ch = pol_v[pl.ds(i * _LANES, _LANES)]
            cs = ch
            for d in (1, 2, 4, 8):
                shifted = _take(cs, jnp.maximum(lanes - d, 0))
                cs = cs + jnp.where(lanes >= d, shifted, 0)
            excl = cs - ch + carry
            idx_v[pl.ds(i * _LANES, _LANES)] = bbase + LV + ch * (excl - LV)
            return carry + _take(cs, jnp.full((_LANES,), _LANES - 1, jnp.int32))

        lax.fori_loop(0, Lp // _LANES, scan_body,
                      jnp.zeros((_LANES,), jnp.int32))

        sets = ((pe0, g0, spe0, sg0, so0), (pe1, g1, spe1, sg1, so1))
        starts = [(h * per_half + k) * _CH for k in range(per_half)]

        def issue_in(k):
            pe_b, g_b, spe, sg, _ = sets[k % 2]
            c1 = pltpu.async_copy(pe_hbm.at[pl.ds(starts[k], _CH)], pe_b, spe)
            c2 = pltpu.async_copy(x_hbm.at[idx_v.at[pl.ds(starts[k], _CH)]],
                                  g_b, sg)
            return c1, c2

        def add_rows(pe_b, g_b, nrows):
            def row_body(r, _):
                def cc_body(q, _2):
                    for t in range(8):
                        sl = pl.ds((q * 8 + t) * _LANES, _LANES)
                        pe_b[r, sl] = pe_b[r, sl] + g_b[r, sl]
                    return 0
                lax.fori_loop(0, ccq, cc_body, 0)
                return 0
            lax.fori_loop(0, nrows, row_body, 0)

        out_inflight = [None, None]
        in_h = issue_in(0)
        for k in range(per_half):
            if k + 1 < per_half:
                nxt = (k + 1) % 2
                if out_inflight[nxt] is not None:
                    out_inflight[nxt].wait()
                    out_inflight[nxt] = None
                nxt_h = issue_in(k + 1)
            in_h[0].wait()
            in_h[1].wait()
            pe_b, g_b, _, _, so = sets[k % 2]
            add_rows(pe_b, g_b, _CH)
            out_inflight[k % 2] = pltpu.async_copy(
                pe_b, out_hbm.at[b, pl.ds(starts[k], _CH)], so)
            if k + 1 < per_half:
                in_h = nxt_h
        for o in out_inflight:
            if o is not None:
                o.wait()

        @pl.when(h == 1)
        def _():
            c1 = pltpu.async_copy(pe_hbm.at[pl.ds(tail, 1)],
                                  pe_t.at[pl.ds(0, 1)], spt)
            c2 = pltpu.async_copy(x_hbm.at[idx_v.at[pl.ds(tail, 8)]], g_t, sgt)
            c1.wait()
            c2.wait()
            add_rows(pe_t, g_t, 1)
            pltpu.sync_copy(pe_t.at[pl.ds(0, 1)],
                            out_hbm.at[b, pl.ds(tail, 1)])

    return sc_kernel


def kernel(x, policy, mask_token, pos_embed):
    B, LV, C = x.shape
    L = policy.shape[1]
    Lp = ((L + _LANES - 1) // _LANES) * _LANES

    polp = jnp.pad(policy.astype(jnp.int32), ((0, 0), (0, Lp - L)))
    x_ext = jnp.concatenate(
        [x, jnp.broadcast_to(mask_token, (B, 1, C)).astype(x.dtype)], axis=1
    ).reshape(B * (LV + 1), C)
    pe = pos_embed.reshape(L, C)

    sc = _build_sc_kernel(B, LV, C, L, Lp)
    return sc(x_ext, polp, pe)


# speedup vs baseline: 1.2597x; 1.2597x over previous
import functools

import jax
import jax.numpy as jnp
from jax import lax
from jax.experimental import pallas as pl
from jax.experimental.pallas import tpu as pltpu
from jax.experimental.pallas import tpu_sc as plsc

_NC, _NS = 2, 16
_CH = 32
_LANES = 16


def _build_sc_kernel(B, LV, C, L, Lp):
    TROWS = LV + 1
    n_chunks = L // _CH
    per_half = n_chunks // 2
    tail = n_chunks * _CH
    mesh = plsc.VectorSubcoreMesh(core_axis_name="c", subcore_axis_name="s")

    @functools.partial(
        pl.kernel,
        mesh=mesh,
        out_type=jax.ShapeDtypeStruct((B, L, C), jnp.float32),
        scratch_types=[
            pltpu.VMEM((Lp,), jnp.int32),
            pltpu.VMEM((Lp,), jnp.int32),
            pltpu.VMEM((_CH, C), jnp.float32),
            pltpu.VMEM((_CH, C), jnp.float32),
            pltpu.VMEM((_CH, C), jnp.float32),
            pltpu.VMEM((_CH, C), jnp.float32),
            pltpu.VMEM((8, C), jnp.float32),
            pltpu.VMEM((8, C), jnp.float32),
            pltpu.SemaphoreType.DMA,
            pltpu.SemaphoreType.DMA,
            pltpu.SemaphoreType.DMA,
            pltpu.SemaphoreType.DMA,
            pltpu.SemaphoreType.DMA,
            pltpu.SemaphoreType.DMA,
            pltpu.SemaphoreType.DMA,
            pltpu.SemaphoreType.DMA,
        ],
    )
    def sc_kernel(x_hbm, pol_hbm, pe_hbm, out_hbm, pol_v, idx_v,
                  pe0, g0, pe1, g1, pe_t, g_t,
                  spe0, sg0, so0, spe1, sg1, so1, spt, sgt):
        wid = lax.axis_index("s") * _NC + lax.axis_index("c")
        b = wid // 2
        h = wid % 2

        pltpu.sync_copy(pol_hbm.at[b], pol_v)

        sets = ((pe0, g0, spe0, sg0, so0), (pe1, g1, spe1, sg1, so1))
        starts = [(h * per_half + k) * _CH for k in range(per_half)]

        def issue_pe(k):
            pe_b, _, spe, _, _ = sets[k % 2]
            return pltpu.async_copy(pe_hbm.at[pl.ds(starts[k], _CH)], pe_b,
                                    spe)

        pe_pre = issue_pe(0)

        bbase = b * TROWS
        lanes = lax.iota(jnp.int32, _LANES)

        def _take(v, i):
            return v.at[i].get(mode="promise_in_bounds")

        def scan_body(i, carry):
            ch = pol_v[pl.ds(i * _LANES, _LANES)]
            cs = ch
            for d in (1, 2, 4, 8):
                shifted = _take(cs, jnp.maximum(lanes - d, 0))
                cs = cs + jnp.where(lanes >= d, shifted, 0)
            excl = cs - ch + carry
            idx_v[pl.ds(i * _LANES, _LANES)] = bbase + LV + ch * (excl - LV)
            return carry + _take(cs, jnp.full((_LANES,), _LANES - 1, jnp.int32))

        lax.fori_loop(0, Lp // _LANES, scan_body,
                      jnp.zeros((_LANES,), jnp.int32))

        def issue_g(k):
            _, g_b, _, sg, _ = sets[k % 2]
            return pltpu.async_copy(x_hbm.at[idx_v.at[pl.ds(starts[k], _CH)]],
                                    g_b, sg)

        def add_rows(pe_b, g_b, nrows):
            def row_body(r, _):
                for cc in range(C // _LANES):
                    sl = pl.ds(cc * _LANES, _LANES)
                    pe_b[r, sl] = pe_b[r, sl] + g_b[r, sl]
                return 0
            lax.fori_loop(0, nrows, row_body, 0)

        out_inflight = [None, None]
        in_h = (pe_pre, issue_g(0))
        for k in range(per_half):
            if k + 1 < per_half:
                nxt = (k + 1) % 2
                if out_inflight[nxt] is not None:
                    out_inflight[nxt].wait()
                    out_inflight[nxt] = None
                nxt_h = (issue_pe(k + 1), issue_g(k + 1))
            in_h[0].wait()
            in_h[1].wait()
            pe_b, g_b, _, _, so = sets[k % 2]
            add_rows(pe_b, g_b, _CH)
            out_inflight[k % 2] = pltpu.async_copy(
                pe_b, out_hbm.at[b, pl.ds(starts[k], _CH)], so)
            if k + 1 < per_half:
                in_h = nxt_h
        for o in out_inflight:
            if o is not None:
                o.wait()

        @pl.when(h == 1)
        def _():
            c1 = pltpu.async_copy(pe_hbm.at[pl.ds(tail, 1)],
                                  pe_t.at[pl.ds(0, 1)], spt)
            c2 = pltpu.async_copy(x_hbm.at[idx_v.at[pl.ds(tail, 8)]], g_t, sgt)
            c1.wait()
            c2.wait()
            add_rows(pe_t, g_t, 1)
            pltpu.sync_copy(pe_t.at[pl.ds(0, 1)],
                            out_hbm.at[b, pl.ds(tail, 1)])

    return sc_kernel


def kernel(x, policy, mask_token, pos_embed):
    B, LV, C = x.shape
    L = policy.shape[1]
    Lp = ((L + _LANES - 1) // _LANES) * _LANES

    polp = jnp.pad(policy.astype(jnp.int32), ((0, 0), (0, Lp - L)))
    x_ext = jnp.concatenate(
        [x, jnp.broadcast_to(mask_token, (B, 1, C))], axis=1
    ).reshape(B * (LV + 1), C)
    pe = pos_embed.reshape(L, C)

    sc = _build_sc_kernel(B, LV, C, L, Lp)
    return sc(x_ext, polp, pe)
